# baseline (speedup 1.0000x reference)
import functools

import jax
import jax.numpy as jnp
from jax import lax
from jax.experimental import pallas as pl
from jax.experimental.pallas import tpu as pltpu
from jax.experimental.pallas import tpu_sc as plsc

E = 8
K = 2
N = 2048
D = 768
F = 2048
T = 256
P = N * K
S = ((P + E * (T - 1) + T - 1) // T) * T
NB = S // T

NC, NS = 2, 16
NW = NC * NS


def _router_body(x_ref, wr_ref, w_ref, i_ref):
    logits = jnp.dot(x_ref[...], wr_ref[...], preferred_element_type=jnp.float32)
    m = jnp.max(logits, axis=-1, keepdims=True)
    p = jnp.exp(logits - m)
    p = p / jnp.sum(p, axis=-1, keepdims=True)
    iota = lax.broadcasted_iota(jnp.int32, p.shape, 1)
    m1 = jnp.max(p, axis=-1, keepdims=True)
    i1 = jnp.min(jnp.where(p == m1, iota, E), axis=-1, keepdims=True)
    p2 = jnp.where(iota == i1, -1.0, p)
    m2 = jnp.max(p2, axis=-1, keepdims=True)
    i2 = jnp.min(jnp.where(p2 == m2, iota, E), axis=-1, keepdims=True)
    s = m1 + m2
    w_ref[...] = jnp.concatenate([m1 / s, m2 / s], axis=1)
    i_ref[...] = jnp.concatenate([i1, i2], axis=1)


def _router(x_flat, Wr):
    return pl.pallas_call(
        _router_body,
        out_shape=(
            jax.ShapeDtypeStruct((N, K), jnp.float32),
            jax.ShapeDtypeStruct((N, K), jnp.int32),
        ),
    )(x_flat, Wr)


GCH = 64


@functools.cache
def _sc_gather_kernel():
    mesh = plsc.VectorSubcoreMesh(
        core_axis_name="c", subcore_axis_name="s", num_cores=NC, num_subcores=NS
    )
    rows_per_w = S // NW
    nch = rows_per_w // GCH
    stage_rows = N // NS

    @functools.partial(
        pl.kernel,
        mesh=mesh,
        out_type=jax.ShapeDtypeStruct((S, D), jnp.float32),
        scratch_types=[
            pltpu.VMEM((rows_per_w,), jnp.int32),
            pltpu.VMEM((GCH, D), jnp.float32),
            pltpu.VMEM((GCH, D), jnp.float32),
            pltpu.SemaphoreType.DMA,
            pltpu.SemaphoreType.DMA,
        ],
    )
    def k(x_hbm, tok_hbm, out_hbm, idx_v, rows_a, rows_b, gsem, wsem):
        sid = lax.axis_index("s")
        wid = sid * NC + lax.axis_index("c")
        base = pl.multiple_of(wid * rows_per_w, 8)
        pltpu.sync_copy(tok_hbm.at[pl.ds(base, rows_per_w)], idx_v)

        bufs = [rows_a, rows_b]
        copies = [None] * nch
        for c in range(2):
            copies[c] = pltpu.async_copy(
                x_hbm.at[idx_v.at[pl.ds(c * GCH, GCH)]], bufs[c % 2], gsem)
        for c in range(nch):
            start = pl.multiple_of(base + c * GCH, 8)
            copies[c].wait()
            pltpu.async_copy(bufs[c % 2], out_hbm.at[pl.ds(start, GCH)],
                             wsem).wait()
            if c + 2 < nch:
                copies[c + 2] = pltpu.async_copy(
                    x_hbm.at[idx_v.at[pl.ds((c + 2) * GCH, GCH)]],
                    bufs[c % 2], gsem)

    return k


def _mlp_body(be_ref, xs_ref, w1_ref, b1_ref, w2_ref, b2_ref, ws_ref, ys_ref):
    h = jax.nn.gelu(
        jnp.dot(xs_ref[...], w1_ref[0], preferred_element_type=jnp.float32)
        + b1_ref[0]
    )
    y = jnp.dot(h, w2_ref[0], preferred_element_type=jnp.float32)
    ys_ref[...] = (y + b2_ref[0]) * ws_ref[...]


def _grouped_mlp(block_expert, Xs, W1, b1, W2, b2, w_slot):
    grid_spec = pltpu.PrefetchScalarGridSpec(
        num_scalar_prefetch=1,
        grid=(NB,),
        in_specs=[
            pl.BlockSpec((T, D), lambda i, be: (i, 0)),
            pl.BlockSpec((1, D, F), lambda i, be: (be[i], 0, 0)),
            pl.BlockSpec((1, 1, F), lambda i, be: (be[i], 0, 0)),
            pl.BlockSpec((1, F, D), lambda i, be: (be[i], 0, 0)),
            pl.BlockSpec((1, 1, D), lambda i, be: (be[i], 0, 0)),
            pl.BlockSpec((T, 1), lambda i, be: (i, 0)),
        ],
        out_specs=pl.BlockSpec((T, D), lambda i, be: (i, 0)),
    )
    return pl.pallas_call(
        _mlp_body,
        grid_spec=grid_spec,
        out_shape=jax.ShapeDtypeStruct((S, D), jnp.float32),
    )(block_expert, Xs, W1, b1.reshape(E, 1, F), W2, b2.reshape(E, 1, D), w_slot)


CCH = 32


@functools.cache
def _sc_combine_kernel():
    mesh = plsc.VectorSubcoreMesh(
        core_axis_name="c", subcore_axis_name="s", num_cores=NC, num_subcores=NS
    )
    tok_per_w = N // NW

    @functools.partial(
        pl.kernel,
        mesh=mesh,
        out_type=jax.ShapeDtypeStruct((N, D), jnp.float32),
        scratch_types=[
            pltpu.VMEM((CCH,), jnp.int32),
            pltpu.VMEM((CCH,), jnp.int32),
            pltpu.VMEM((CCH, D), jnp.float32),
            pltpu.VMEM((CCH, D), jnp.float32),
            pltpu.SemaphoreType.DMA,
        ],
    )
    def k(ys_hbm, p0_hbm, p1_hbm, out_hbm, i0_v, i1_v, r0_v, r1_v, sem):
        wid = lax.axis_index("s") * NC + lax.axis_index("c")
        base = pl.multiple_of(wid * tok_per_w, CCH)

        def chunk(c, carry):
            start = pl.multiple_of(base + c * CCH, 8)
            pltpu.sync_copy(p0_hbm.at[pl.ds(start, CCH)], i0_v)
            pltpu.sync_copy(p1_hbm.at[pl.ds(start, CCH)], i1_v)
            cp0 = pltpu.async_copy(ys_hbm.at[i0_v], r0_v, sem)
            cp1 = pltpu.async_copy(ys_hbm.at[i1_v], r1_v, sem)
            cp0.wait()
            cp1.wait()

            def row(i, rcarry):
                for ch in range(D // 16):
                    sl = pl.ds(ch * 16, 16)
                    r0_v[i, sl] = r0_v[i, sl] + r1_v[i, sl]
                return rcarry

            lax.fori_loop(0, CCH, row, 0)
            pltpu.sync_copy(r0_v, out_hbm.at[pl.ds(start, CCH)])
            return carry

        lax.fori_loop(0, tok_per_w // CCH, chunk, 0)

    return k


def kernel(x, Wr, W1, b1, W2, b2):
    Bb, Ll, Dd = x.shape
    x_flat = x.reshape(Bb * Ll, Dd)

    w, idx = _router(x_flat, Wr)

    e = idx.reshape(P)
    wf = w.reshape(P)
    oh = (e[:, None] == jnp.arange(E, dtype=jnp.int32)[None, :]).astype(jnp.int32)
    csum = jnp.cumsum(oh, axis=0)
    rank = jnp.take_along_axis(csum, e[:, None], axis=1)[:, 0] - 1
    cnt = csum[-1]
    cnt_pad = ((cnt + T - 1) // T) * T
    pad_cum = jnp.cumsum(cnt_pad)
    pad_off = pad_cum - cnt_pad
    dest = (pad_off[e] + rank).astype(jnp.int32)
    row_token = (
        jnp.zeros((S,), jnp.int32)
        .at[dest].set(jnp.arange(P, dtype=jnp.int32) // K)
    )
    w_slot = jnp.zeros((S, 1), jnp.float32).at[dest, 0].set(wf)
    block_expert = jnp.minimum(
        jnp.searchsorted(pad_cum, jnp.arange(NB, dtype=jnp.int32) * T, side="right"),
        E - 1,
    ).astype(jnp.int32)
    pos = dest.reshape(N, K)
    pos0 = pos[:, 0]
    pos1 = pos[:, 1]

    Xs = _sc_gather_kernel()(x_flat, row_token)
    Ys = _grouped_mlp(block_expert, Xs, W1, b1, W2, b2, w_slot)
    out = _sc_combine_kernel()(Ys, pos0, pos1)
    return out.reshape(Bb, Ll, Dd)


# speedup vs baseline: 1.7957x; 1.7957x over previous
import functools

import jax
import jax.numpy as jnp
from jax import lax
from jax.experimental import pallas as pl
from jax.experimental.pallas import tpu as pltpu
from jax.experimental.pallas import tpu_sc as plsc

E = 8
K = 2
N = 2048
D = 768
F = 2048
T = 256
P = N * K
S = ((P + E * (T - 1) + T - 1) // T) * T
NB = S // T

NC, NS = 2, 16
NW = NC * NS
L = 16


def _router_body(x_ref, wr_ref, i_ref, w0_ref, w1_ref):
    logits = jnp.dot(x_ref[...], wr_ref[...], preferred_element_type=jnp.float32)
    m = jnp.max(logits, axis=-1, keepdims=True)
    p = jnp.exp(logits - m)
    p = p / jnp.sum(p, axis=-1, keepdims=True)
    iota = lax.broadcasted_iota(jnp.int32, p.shape, 1)
    m1 = jnp.max(p, axis=-1, keepdims=True)
    i1 = jnp.min(jnp.where(p == m1, iota, E), axis=-1, keepdims=True)
    p2 = jnp.where(iota == i1, -1.0, p)
    m2 = jnp.max(p2, axis=-1, keepdims=True)
    i2 = jnp.min(jnp.where(p2 == m2, iota, E), axis=-1, keepdims=True)
    s = m1 + m2
    i_ref[...] = jnp.concatenate([i1, i2], axis=1)
    w0_ref[...] = jnp.broadcast_to(m1 / s, (m1.shape[0], L))
    w1_ref[...] = jnp.broadcast_to(m2 / s, (m2.shape[0], L))


def _router(x_flat, Wr):
    return pl.pallas_call(
        _router_body,
        out_shape=(
            jax.ShapeDtypeStruct((N, K), jnp.int32),
            jax.ShapeDtypeStruct((N, L), jnp.float32),
            jax.ShapeDtypeStruct((N, L), jnp.float32),
        ),
    )(x_flat, Wr)


@functools.cache
def _sc_dispatch_kernel():
    mesh = plsc.VectorSubcoreMesh(
        core_axis_name="c", subcore_axis_name="s", num_cores=NC, num_subcores=NS
    )
    tok_per_w = N // NW

    @functools.partial(
        pl.kernel,
        mesh=mesh,
        out_type=jax.ShapeDtypeStruct((S, D), jnp.float32),
        scratch_types=[
            pltpu.VMEM((tok_per_w, D), jnp.float32),
            pltpu.VMEM((tok_per_w,), jnp.int32),
            pltpu.VMEM((tok_per_w,), jnp.int32),
            pltpu.SemaphoreType.DMA,
            pltpu.SemaphoreType.DMA,
        ],
    )
    def k(x_hbm, p0_hbm, p1_hbm, xs_hbm, xbuf, d0_v, d1_v, lsem, ssem):
        wid = lax.axis_index("s") * NC + lax.axis_index("c")
        tbase = pl.multiple_of(wid * tok_per_w, 8)
        lc = pltpu.async_copy(x_hbm.at[pl.ds(tbase, tok_per_w)], xbuf, lsem)
        pltpu.sync_copy(p0_hbm.at[pl.ds(tbase, tok_per_w)], d0_v)
        pltpu.sync_copy(p1_hbm.at[pl.ds(tbase, tok_per_w)], d1_v)
        lc.wait()
        s0 = pltpu.async_copy(xbuf, xs_hbm.at[d0_v], ssem)
        s1 = pltpu.async_copy(xbuf, xs_hbm.at[d1_v], ssem)
        s0.wait()
        s1.wait()

    return k


def _mlp_body(be_ref, xs_ref, w1_ref, b1_ref, w2_ref, b2_ref, ys_ref):
    h = jax.nn.gelu(
        jnp.dot(xs_ref[...], w1_ref[0], preferred_element_type=jnp.float32)
        + b1_ref[0]
    )
    y = jnp.dot(h, w2_ref[0], preferred_element_type=jnp.float32)
    ys_ref[...] = y + b2_ref[0]


def _grouped_mlp(block_expert, Xs, W1, b1, W2, b2):
    grid_spec = pltpu.PrefetchScalarGridSpec(
        num_scalar_prefetch=1,
        grid=(NB,),
        in_specs=[
            pl.BlockSpec((T, D), lambda i, be: (i, 0)),
            pl.BlockSpec((1, D, F), lambda i, be: (be[i], 0, 0)),
            pl.BlockSpec((1, 1, F), lambda i, be: (be[i], 0, 0)),
            pl.BlockSpec((1, F, D), lambda i, be: (be[i], 0, 0)),
            pl.BlockSpec((1, 1, D), lambda i, be: (be[i], 0, 0)),
        ],
        out_specs=pl.BlockSpec((T, D), lambda i, be: (i, 0)),
    )
    return pl.pallas_call(
        _mlp_body,
        grid_spec=grid_spec,
        out_shape=jax.ShapeDtypeStruct((S, D), jnp.float32),
    )(block_expert, Xs, W1, b1.reshape(E, 1, F), W2, b2.reshape(E, 1, D))


CCH = 32


@functools.cache
def _sc_combine_kernel():
    mesh = plsc.VectorSubcoreMesh(
        core_axis_name="c", subcore_axis_name="s", num_cores=NC, num_subcores=NS
    )
    tok_per_w = N // NW

    @functools.partial(
        pl.kernel,
        mesh=mesh,
        out_type=jax.ShapeDtypeStruct((N, D), jnp.float32),
        scratch_types=[
            pltpu.VMEM((CCH,), jnp.int32),
            pltpu.VMEM((CCH,), jnp.int32),
            pltpu.VMEM((CCH, D), jnp.float32),
            pltpu.VMEM((CCH, D), jnp.float32),
            pltpu.VMEM((CCH, L), jnp.float32),
            pltpu.VMEM((CCH, L), jnp.float32),
            pltpu.SemaphoreType.DMA,
        ],
    )
    def k(ys_hbm, p0_hbm, p1_hbm, w0_hbm, w1_hbm, out_hbm,
          i0_v, i1_v, r0_v, r1_v, w0_v, w1_v, sem):
        wid = lax.axis_index("s") * NC + lax.axis_index("c")
        base = pl.multiple_of(wid * tok_per_w, CCH)

        def chunk(c, carry):
            start = pl.multiple_of(base + c * CCH, 8)
            pltpu.sync_copy(p0_hbm.at[pl.ds(start, CCH)], i0_v)
            pltpu.sync_copy(p1_hbm.at[pl.ds(start, CCH)], i1_v)
            cp0 = pltpu.async_copy(ys_hbm.at[i0_v], r0_v, sem)
            cp1 = pltpu.async_copy(ys_hbm.at[i1_v], r1_v, sem)
            pltpu.sync_copy(w0_hbm.at[pl.ds(start, CCH)], w0_v)
            pltpu.sync_copy(w1_hbm.at[pl.ds(start, CCH)], w1_v)
            cp0.wait()
            cp1.wait()

            def row(i, rcarry):
                wa = w0_v[i]
                wb = w1_v[i]
                for ch in range(D // L):
                    sl = pl.ds(ch * L, L)
                    r0_v[i, sl] = r0_v[i, sl] * wa + r1_v[i, sl] * wb
                return rcarry

            lax.fori_loop(0, CCH, row, 0)
            pltpu.sync_copy(r0_v, out_hbm.at[pl.ds(start, CCH)])
            return carry

        lax.fori_loop(0, tok_per_w // CCH, chunk, 0)

    return k


def kernel(x, Wr, W1, b1, W2, b2):
    Bb, Ll, Dd = x.shape
    x_flat = x.reshape(Bb * Ll, Dd)

    idx, w0b, w1b = _router(x_flat, Wr)

    e = idx.reshape(P)
    oh = (e[:, None] == jnp.arange(E, dtype=jnp.int32)[None, :]).astype(jnp.int32)
    csum = jnp.cumsum(oh, axis=0)
    rank = jnp.take_along_axis(csum, e[:, None], axis=1)[:, 0] - 1
    cnt = csum[-1]
    cnt_pad = ((cnt + T - 1) // T) * T
    pad_cum = jnp.cumsum(cnt_pad)
    pad_off = pad_cum - cnt_pad
    dest = (pad_off[e] + rank).astype(jnp.int32)
    block_expert = jnp.minimum(
        jnp.searchsorted(pad_cum, jnp.arange(NB, dtype=jnp.int32) * T, side="right"),
        E - 1,
    ).astype(jnp.int32)
    pos = dest.reshape(N, K)
    pos0 = pos[:, 0]
    pos1 = pos[:, 1]

    Xs = _sc_dispatch_kernel()(x_flat, pos0, pos1)
    Ys = _grouped_mlp(block_expert, Xs, W1, b1, W2, b2)
    out = _sc_combine_kernel()(Ys, pos0, pos1, w0b, w1b)
    return out.reshape(Bb, Ll, Dd)


# speedup vs baseline: 1.8567x; 1.0340x over previous
import functools

import jax
import jax.numpy as jnp
from jax import lax
from jax.experimental import pallas as pl
from jax.experimental.pallas import tpu as pltpu
from jax.experimental.pallas import tpu_sc as plsc

E = 8
K = 2
N = 2048
D = 768
F = 2048
T = 256
P = N * K
S = ((P + E * (T - 1) + T - 1) // T) * T
NB = S // T

NC, NS = 2, 16
NW = NC * NS
L = 16


def _router_body(x_ref, wr_ref, i_ref, w0_ref, w1_ref):
    logits = jnp.dot(x_ref[...], wr_ref[...], preferred_element_type=jnp.float32)
    m = jnp.max(logits, axis=-1, keepdims=True)
    p = jnp.exp(logits - m)
    p = p / jnp.sum(p, axis=-1, keepdims=True)
    iota = lax.broadcasted_iota(jnp.int32, p.shape, 1)
    m1 = jnp.max(p, axis=-1, keepdims=True)
    i1 = jnp.min(jnp.where(p == m1, iota, E), axis=-1, keepdims=True)
    p2 = jnp.where(iota == i1, -1.0, p)
    m2 = jnp.max(p2, axis=-1, keepdims=True)
    i2 = jnp.min(jnp.where(p2 == m2, iota, E), axis=-1, keepdims=True)
    s = m1 + m2
    i_ref[...] = jnp.concatenate([i1, i2], axis=1)
    w0_ref[...] = jnp.broadcast_to(m1 / s, (m1.shape[0], L))
    w1_ref[...] = jnp.broadcast_to(m2 / s, (m2.shape[0], L))


def _router(x_flat, Wr):
    return pl.pallas_call(
        _router_body,
        out_shape=(
            jax.ShapeDtypeStruct((N, K), jnp.int32),
            jax.ShapeDtypeStruct((N, L), jnp.float32),
            jax.ShapeDtypeStruct((N, L), jnp.float32),
        ),
    )(x_flat, Wr)


@functools.cache
def _sc_dispatch_kernel():
    mesh = plsc.VectorSubcoreMesh(
        core_axis_name="c", subcore_axis_name="s", num_cores=NC, num_subcores=NS
    )
    tok_per_w = N // NW

    @functools.partial(
        pl.kernel,
        mesh=mesh,
        out_type=jax.ShapeDtypeStruct((S, D), jnp.float32),
        scratch_types=[
            pltpu.VMEM((tok_per_w, D), jnp.float32),
            pltpu.VMEM((tok_per_w,), jnp.int32),
            pltpu.VMEM((tok_per_w,), jnp.int32),
            pltpu.SemaphoreType.DMA,
            pltpu.SemaphoreType.DMA,
        ],
    )
    def k(x_hbm, p0_hbm, p1_hbm, xs_hbm, xbuf, d0_v, d1_v, lsem, ssem):
        wid = lax.axis_index("s") * NC + lax.axis_index("c")
        tbase = pl.multiple_of(wid * tok_per_w, 8)
        lc = pltpu.async_copy(x_hbm.at[pl.ds(tbase, tok_per_w)], xbuf, lsem)
        pltpu.sync_copy(p0_hbm.at[pl.ds(tbase, tok_per_w)], d0_v)
        pltpu.sync_copy(p1_hbm.at[pl.ds(tbase, tok_per_w)], d1_v)
        lc.wait()
        s0 = pltpu.async_copy(xbuf, xs_hbm.at[d0_v], ssem)
        s1 = pltpu.async_copy(xbuf, xs_hbm.at[d1_v], ssem)
        s0.wait()
        s1.wait()

    return k


def _mlp_body(be_ref, xs_ref, w1_ref, b1_ref, w2_ref, b2_ref, ys_ref):
    i = pl.program_id(0)

    @pl.when(i < be_ref[NB])
    def _():
        h = jax.nn.gelu(
            jnp.dot(xs_ref[...], w1_ref[0], preferred_element_type=jnp.float32)
            + b1_ref[0]
        )
        y = jnp.dot(h, w2_ref[0], preferred_element_type=jnp.float32)
        ys_ref[...] = y + b2_ref[0]


def _grouped_mlp(block_expert, Xs, W1, b1, W2, b2):
    grid_spec = pltpu.PrefetchScalarGridSpec(
        num_scalar_prefetch=1,
        grid=(NB,),
        in_specs=[
            pl.BlockSpec((T, D), lambda i, be: (i, 0)),
            pl.BlockSpec((1, D, F), lambda i, be: (be[i], 0, 0)),
            pl.BlockSpec((1, 1, F), lambda i, be: (be[i], 0, 0)),
            pl.BlockSpec((1, F, D), lambda i, be: (be[i], 0, 0)),
            pl.BlockSpec((1, 1, D), lambda i, be: (be[i], 0, 0)),
        ],
        out_specs=pl.BlockSpec((T, D), lambda i, be: (i, 0)),
    )
    return pl.pallas_call(
        _mlp_body,
        grid_spec=grid_spec,
        out_shape=jax.ShapeDtypeStruct((S, D), jnp.float32),
    )(block_expert, Xs, W1, b1.reshape(E, 1, F), W2, b2.reshape(E, 1, D))


CCH = 32


@functools.cache
def _sc_combine_kernel():
    mesh = plsc.VectorSubcoreMesh(
        core_axis_name="c", subcore_axis_name="s", num_cores=NC, num_subcores=NS
    )
    tok_per_w = N // NW

    @functools.partial(
        pl.kernel,
        mesh=mesh,
        out_type=jax.ShapeDtypeStruct((N, D), jnp.float32),
        scratch_types=[
            pltpu.VMEM((CCH,), jnp.int32),
            pltpu.VMEM((CCH,), jnp.int32),
            pltpu.VMEM((CCH, D), jnp.float32),
            pltpu.VMEM((CCH, D), jnp.float32),
            pltpu.VMEM((CCH, D), jnp.float32),
            pltpu.VMEM((CCH, L), jnp.float32),
            pltpu.VMEM((CCH, L), jnp.float32),
            pltpu.SemaphoreType.DMA,
        ],
    )
    def k(ys_hbm, p0_hbm, p1_hbm, w0_hbm, w1_hbm, out_hbm,
          i0_v, i1_v, r0_v, r1_v, obuf, w0_v, w1_v, sem):
        wid = lax.axis_index("s") * NC + lax.axis_index("c")
        base = pl.multiple_of(wid * tok_per_w, CCH)

        def chunk(c, carry):
            start = pl.multiple_of(base + c * CCH, 8)
            pltpu.sync_copy(p0_hbm.at[pl.ds(start, CCH)], i0_v)
            pltpu.sync_copy(p1_hbm.at[pl.ds(start, CCH)], i1_v)
            cp0 = pltpu.async_copy(ys_hbm.at[i0_v], r0_v, sem)
            cp1 = pltpu.async_copy(ys_hbm.at[i1_v], r1_v, sem)
            pltpu.sync_copy(w0_hbm.at[pl.ds(start, CCH)], w0_v)
            pltpu.sync_copy(w1_hbm.at[pl.ds(start, CCH)], w1_v)
            cp0.wait()
            cp1.wait()

            def row(i, rcarry):
                wa = w0_v[i]
                wb = w1_v[i]
                for ch in range(D // L):
                    sl = pl.ds(ch * L, L)
                    obuf[i, sl] = r0_v[i, sl] * wa + r1_v[i, sl] * wb
                return rcarry

            lax.fori_loop(0, CCH, row, 0)
            pltpu.sync_copy(obuf, out_hbm.at[pl.ds(start, CCH)])
            return carry

        lax.fori_loop(0, tok_per_w // CCH, chunk, 0)

    return k


def kernel(x, Wr, W1, b1, W2, b2):
    Bb, Ll, Dd = x.shape
    x_flat = x.reshape(Bb * Ll, Dd)

    idx, w0b, w1b = _router(x_flat, Wr)

    e = idx.reshape(P)
    oh = (e[:, None] == jnp.arange(E, dtype=jnp.int32)[None, :]).astype(jnp.int32)
    csum = jnp.cumsum(oh, axis=0)
    rank = jnp.take_along_axis(csum, e[:, None], axis=1)[:, 0] - 1
    cnt = csum[-1]
    cnt_pad = ((cnt + T - 1) // T) * T
    pad_cum = jnp.cumsum(cnt_pad)
    pad_off = pad_cum - cnt_pad
    dest = (pad_off[e] + rank).astype(jnp.int32)
    block_expert = jnp.minimum(
        jnp.searchsorted(pad_cum, jnp.arange(NB, dtype=jnp.int32) * T, side="right"),
        E - 1,
    ).astype(jnp.int32)
    used_blocks = (pad_cum[-1] // T).astype(jnp.int32)
    block_expert = jnp.concatenate([block_expert, used_blocks[None]])
    pos = dest.reshape(N, K)
    pos0 = pos[:, 0]
    pos1 = pos[:, 1]

    Xs = _sc_dispatch_kernel()(x_flat, pos0, pos1)
    Ys = _grouped_mlp(block_expert, Xs, W1, b1, W2, b2)
    out = _sc_combine_kernel()(Ys, pos0, pos1, w0b, w1b)
    return out.reshape(Bb, Ll, Dd)


# speedup vs baseline: 2.0541x; 1.1063x over previous
import functools

import jax
import jax.numpy as jnp
from jax import lax
from jax.experimental import pallas as pl
from jax.experimental.pallas import tpu as pltpu
from jax.experimental.pallas import tpu_sc as plsc

E = 8
K = 2
N = 2048
D = 768
F = 2048
T = 256
P = N * K
S = ((P + E * (T - 1) + T - 1) // T) * T
NB = S // T

NC, NS = 2, 16
NW = NC * NS
L = 16


def _router_body(x_ref, wr_ref, i_ref, w0_ref, w1_ref):
    logits = jnp.dot(x_ref[...], wr_ref[...], preferred_element_type=jnp.float32)
    m = jnp.max(logits, axis=-1, keepdims=True)
    p = jnp.exp(logits - m)
    p = p / jnp.sum(p, axis=-1, keepdims=True)
    iota = lax.broadcasted_iota(jnp.int32, p.shape, 1)
    m1 = jnp.max(p, axis=-1, keepdims=True)
    i1 = jnp.min(jnp.where(p == m1, iota, E), axis=-1, keepdims=True)
    p2 = jnp.where(iota == i1, -1.0, p)
    m2 = jnp.max(p2, axis=-1, keepdims=True)
    i2 = jnp.min(jnp.where(p2 == m2, iota, E), axis=-1, keepdims=True)
    s = m1 + m2
    i_ref[...] = jnp.concatenate([i1, i2], axis=1)
    w0_ref[...] = jnp.broadcast_to(m1 / s, (m1.shape[0], L))
    w1_ref[...] = jnp.broadcast_to(m2 / s, (m2.shape[0], L))


def _router(x_flat, Wr):
    return pl.pallas_call(
        _router_body,
        out_shape=(
            jax.ShapeDtypeStruct((N, K), jnp.int32),
            jax.ShapeDtypeStruct((N, L), jnp.float32),
            jax.ShapeDtypeStruct((N, L), jnp.float32),
        ),
    )(x_flat, Wr)


@functools.cache
def _sc_dispatch_kernel():
    mesh = plsc.VectorSubcoreMesh(
        core_axis_name="c", subcore_axis_name="s", num_cores=NC, num_subcores=NS
    )
    tok_per_w = N // NW

    @functools.partial(
        pl.kernel,
        mesh=mesh,
        out_type=jax.ShapeDtypeStruct((S, D), jnp.float32),
        scratch_types=[
            pltpu.VMEM((tok_per_w, D), jnp.float32),
            pltpu.VMEM((tok_per_w,), jnp.int32),
            pltpu.VMEM((tok_per_w,), jnp.int32),
            pltpu.SemaphoreType.DMA,
            pltpu.SemaphoreType.DMA,
        ],
    )
    def k(x_hbm, p0_hbm, p1_hbm, xs_hbm, xbuf, d0_v, d1_v, lsem, ssem):
        wid = lax.axis_index("s") * NC + lax.axis_index("c")
        tbase = pl.multiple_of(wid * tok_per_w, 8)
        lc = pltpu.async_copy(x_hbm.at[pl.ds(tbase, tok_per_w)], xbuf, lsem)
        pltpu.sync_copy(p0_hbm.at[pl.ds(tbase, tok_per_w)], d0_v)
        pltpu.sync_copy(p1_hbm.at[pl.ds(tbase, tok_per_w)], d1_v)
        lc.wait()
        s0 = pltpu.async_copy(xbuf, xs_hbm.at[d0_v], ssem)
        s1 = pltpu.async_copy(xbuf, xs_hbm.at[d1_v], ssem)
        s0.wait()
        s1.wait()

    return k


def _mlp_body(be_ref, xs_ref, w1_ref, b1_ref, w2_ref, b2_ref, ys_ref):
    i = pl.program_id(0)

    @pl.when(i < be_ref[NB])
    def _():
        h = jax.nn.gelu(
            jnp.dot(xs_ref[...], w1_ref[0], preferred_element_type=jnp.float32)
            + b1_ref[0]
        )
        y = jnp.dot(h, w2_ref[0], preferred_element_type=jnp.float32)
        ys_ref[...] = y + b2_ref[0]


def _grouped_mlp(block_expert, Xs, W1, b1, W2, b2):
    grid_spec = pltpu.PrefetchScalarGridSpec(
        num_scalar_prefetch=1,
        grid=(NB,),
        in_specs=[
            pl.BlockSpec((T, D), lambda i, be: (i, 0)),
            pl.BlockSpec((1, D, F), lambda i, be: (be[i], 0, 0)),
            pl.BlockSpec((1, 1, F), lambda i, be: (be[i], 0, 0)),
            pl.BlockSpec((1, F, D), lambda i, be: (be[i], 0, 0)),
            pl.BlockSpec((1, 1, D), lambda i, be: (be[i], 0, 0)),
        ],
        out_specs=pl.BlockSpec((T, D), lambda i, be: (i, 0)),
    )
    return pl.pallas_call(
        _mlp_body,
        grid_spec=grid_spec,
        out_shape=jax.ShapeDtypeStruct((S, D), jnp.float32),
    )(block_expert, Xs, W1, b1.reshape(E, 1, F), W2, b2.reshape(E, 1, D))


CCH = 32


@functools.cache
def _sc_combine_kernel():
    mesh = plsc.VectorSubcoreMesh(
        core_axis_name="c", subcore_axis_name="s", num_cores=NC, num_subcores=NS
    )
    tok_per_w = N // NW

    @functools.partial(
        pl.kernel,
        mesh=mesh,
        out_type=jax.ShapeDtypeStruct((N, D), jnp.float32),
        scratch_types=[
            pltpu.VMEM((CCH,), jnp.int32),
            pltpu.VMEM((CCH,), jnp.int32),
            pltpu.VMEM((CCH, D), jnp.float32),
            pltpu.VMEM((CCH, D), jnp.float32),
            pltpu.VMEM((CCH, D), jnp.float32),
            pltpu.VMEM((CCH, L), jnp.float32),
            pltpu.VMEM((CCH, L), jnp.float32),
            pltpu.SemaphoreType.DMA,
        ],
    )
    def k(ys_hbm, p0_hbm, p1_hbm, w0_hbm, w1_hbm, out_hbm,
          i0_v, i1_v, r0_v, r1_v, obuf, w0_v, w1_v, sem):
        wid = lax.axis_index("s") * NC + lax.axis_index("c")
        base = pl.multiple_of(wid * tok_per_w, CCH)

        def chunk(c, carry):
            start = pl.multiple_of(base + c * CCH, 8)
            pltpu.sync_copy(p0_hbm.at[pl.ds(start, CCH)], i0_v)
            pltpu.sync_copy(p1_hbm.at[pl.ds(start, CCH)], i1_v)
            cp0 = pltpu.async_copy(ys_hbm.at[i0_v], r0_v, sem)
            cp1 = pltpu.async_copy(ys_hbm.at[i1_v], r1_v, sem)
            pltpu.sync_copy(w0_hbm.at[pl.ds(start, CCH)], w0_v)
            pltpu.sync_copy(w1_hbm.at[pl.ds(start, CCH)], w1_v)
            cp0.wait()
            cp1.wait()

            def row(i, rcarry):
                wa = w0_v[i]
                wb = w1_v[i]
                for ch in range(D // L):
                    sl = pl.ds(ch * L, L)
                    obuf[i, sl] = r0_v[i, sl] * wa + r1_v[i, sl] * wb
                return rcarry

            lax.fori_loop(0, CCH, row, 0)
            pltpu.sync_copy(obuf, out_hbm.at[pl.ds(start, CCH)])
            return carry

        lax.fori_loop(0, tok_per_w // CCH, chunk, 0)

    return k


def kernel(x, Wr, W1, b1, W2, b2):
    Bb, Ll, Dd = x.shape
    x_flat = x.reshape(Bb * Ll, Dd)

    idx, w0b, w1b = _router(x_flat, Wr)

    iota_e = jnp.arange(E, dtype=jnp.int32)[None, :]
    oh1 = (idx[:, 0:1] == iota_e).astype(jnp.int32)
    oh2 = (idx[:, 1:2] == iota_e).astype(jnp.int32)
    ohf = oh1 + oh2
    c_incl = jnp.cumsum(ohf, axis=0)
    c_excl = c_incl - ohf
    cnt = c_incl[-1]
    cnt_pad = ((cnt + T - 1) // T) * T
    pad_cum = jnp.cumsum(cnt_pad)
    pad_off = (pad_cum - cnt_pad)[None, :]
    pos0 = jnp.sum(oh1 * (pad_off + c_excl), axis=-1, dtype=jnp.int32)
    pos1 = jnp.sum(oh2 * (pad_off + c_excl + oh1), axis=-1, dtype=jnp.int32)
    block_expert = jnp.minimum(
        jnp.searchsorted(pad_cum, jnp.arange(NB, dtype=jnp.int32) * T, side="right"),
        E - 1,
    ).astype(jnp.int32)
    used_blocks = (pad_cum[-1] // T).astype(jnp.int32)
    block_expert = jnp.concatenate([block_expert, used_blocks[None]])

    Xs = _sc_dispatch_kernel()(x_flat, pos0, pos1)
    Ys = _grouped_mlp(block_expert, Xs, W1, b1, W2, b2)
    out = _sc_combine_kernel()(Ys, pos0, pos1, w0b, w1b)
    return out.reshape(Bb, Ll, Dd)
